# Initial kernel scaffold; baseline (speedup 1.0000x reference)
import jax
import jax.numpy as jnp
from jax.experimental import pallas as pl


def kernel(embeddings_a, embeddings_b, R, b):
    raise NotImplementedError("write your pallas kernel here")


# speedup vs baseline: 3.1338x; 3.1338x over previous
import jax
import jax.numpy as jnp
import numpy as np
from jax.experimental import pallas as pl
from jax.experimental.pallas import tpu as pltpu


def _fused_body(b_ref, a_ref, bemb_ref, r_ref, out_ref, rb_ref, bias_ref):
    i = pl.program_id(1)

    @pl.when(i == 0)
    def _per_batch_init():
        bemb = bemb_ref[0]
        rb_ref[...] = jnp.dot(bemb, r_ref[...], preferred_element_type=jnp.float32)
        ones = jnp.ones((8, bemb.shape[1]), dtype=jnp.float32)
        s = jax.lax.dot_general(
            ones, jnp.abs(bemb), (((1,), (1,)), ((), ())),
            preferred_element_type=jnp.float32)
        bias_ref[...] = jnp.where(s == 0.0, -1e9, 0.0) + b_ref[0]

    d = r_ref.shape[0]
    rd = r_ref.shape[1]
    scale = np.float32(1.0 / (d * np.sqrt(rd)))
    ra = jnp.dot(a_ref[0], r_ref[...], preferred_element_type=jnp.float32) * scale
    scores = jax.lax.dot_general(
        ra, rb_ref[...], (((1,), (1,)), ((), ())),
        preferred_element_type=jnp.float32)
    scores = scores + bias_ref[0:1, :]
    m = jnp.max(scores, axis=-1, keepdims=True)
    e = jnp.exp(scores - m)
    ssum = jnp.sum(e, axis=-1, keepdims=True)
    out_ref[0] = e / ssum


def kernel(embeddings_a, embeddings_b, R, b):
    batch, k1, d = embeddings_a.shape
    k2 = embeddings_b.shape[1]
    rd = R.shape[1]
    blk = 512 if k1 % 512 == 0 else k1

    return pl.pallas_call(
        _fused_body,
        grid=(batch, k1 // blk),
        in_specs=[
            pl.BlockSpec(memory_space=pltpu.SMEM),
            pl.BlockSpec((1, blk, d), lambda bi, i: (bi, i, 0)),
            pl.BlockSpec((1, k2, d), lambda bi, i: (bi, 0, 0)),
            pl.BlockSpec((d, rd), lambda bi, i: (0, 0)),
        ],
        out_specs=pl.BlockSpec((1, blk, k2), lambda bi, i: (bi, i, 0)),
        out_shape=jax.ShapeDtypeStruct((batch, k1, k2), jnp.float32),
        scratch_shapes=[
            pltpu.VMEM((k2, rd), jnp.float32),
            pltpu.VMEM((8, k2), jnp.float32),
        ],
        compiler_params=pltpu.CompilerParams(
            dimension_semantics=("parallel", "arbitrary"),
            vmem_limit_bytes=48 * 1024 * 1024,
        ),
        name="fused_bilinear_softmax",
    )(b, embeddings_a, embeddings_b, R)
